# baseline (speedup 1.0000x reference)
import functools

import jax
import jax.numpy as jnp
from jax import lax
from jax.experimental import pallas as pl
from jax.experimental.pallas import tpu as pltpu
from jax.experimental.pallas import tpu_sc as plsc

N_EMB = 8192
N_REL = 4096
D = 1024
L = 16
INV_SQRT_D = 1.0 / (D ** 0.5)


_PROJ_G = 4
_PROJ_B = N_EMB // _PROJ_G


def _proj_body(q_ref, w_ref, e_ref, p_ref, v_scr):
    @pl.when(pl.program_id(0) == 0)
    def _():
        q2d = q_ref[...].reshape(1, D)
        v_scr[...] = lax.dot_general(
            q2d, w_ref[...], (((1,), (1,)), ((), ())),
            preferred_element_type=jnp.float32,
            precision=lax.Precision.DEFAULT)

    pb = lax.dot_general(
        v_scr[...], e_ref[...], (((1,), (1,)), ((), ())),
        preferred_element_type=jnp.float32,
        precision=lax.Precision.DEFAULT)
    p_ref[...] = pb.reshape(_PROJ_B)


_proj = pl.pallas_call(
    _proj_body,
    grid=(_PROJ_G,),
    in_specs=[
        pl.BlockSpec((D,), lambda i: (0,)),
        pl.BlockSpec((D, D), lambda i: (0, 0)),
        pl.BlockSpec((_PROJ_B, D), lambda i: (i, 0)),
    ],
    out_specs=pl.BlockSpec((_PROJ_B,), lambda i: (i,)),
    out_shape=jax.ShapeDtypeStruct((N_EMB,), jnp.float32),
    scratch_shapes=[pltpu.VMEM((1, D), jnp.float32)],
)


_sc_mesh = plsc.VectorSubcoreMesh(
    core_axis_name="c", subcore_axis_name="s", num_cores=2, num_subcores=16)

_NT = 16
_EPT = N_EMB // _NT
_RPT = N_REL // _NT
_CHUNK = 128


@functools.partial(
    pl.kernel,
    out_type=jax.ShapeDtypeStruct((N_EMB,), jnp.float32),
    mesh=_sc_mesh,
    scratch_types=[
        pltpu.VMEM((_EPT // _CHUNK, _CHUNK), jnp.float32),
        pltpu.VMEM((_EPT // _CHUNK, _CHUNK), jnp.int32),
        pltpu.VMEM((_EPT // _CHUNK, _CHUNK), jnp.float32),
        pltpu.VMEM((_RPT,), jnp.float32),
        pltpu.VMEM((_RPT,), jnp.float32),
        pltpu.VMEM((L,), jnp.float32),
        pltpu.VMEM((_NT * L,), jnp.float32),
        pltpu.VMEM((N_REL,), jnp.float32),
        pltpu.VMEM((_EPT,), jnp.float32),
        pltpu.VMEM_SHARED((N_REL,), jnp.float32),
        pltpu.VMEM_SHARED((N_REL,), jnp.float32),
        pltpu.VMEM_SHARED((N_REL,), jnp.float32),
        pltpu.VMEM_SHARED((_NT * L,), jnp.float32),
        pltpu.VMEM_SHARED((_NT * L,), jnp.float32),
        pltpu.SemaphoreType.DMA,
        pltpu.SemaphoreType.DMA,
        pltpu.SemaphoreType.DMA,
    ],
    compiler_params=pltpu.CompilerParams(needs_layout_passes=False),
)
def _sc_middle(p_hbm, idx_hbm, coeff_hbm,
               p_v, idx_v, ones_v, seg_v, cnt_v, part_v, red_v, w_v, o_v,
               s_sh, c_sh, w_sh, mx_sh, sm_sh, sem_p, sem_i, sem_sc):
    cid = lax.axis_index("c")
    sid = lax.axis_index("s")

    @pl.when(cid == 0)
    def _():
        tid = sid
        zeros = jnp.zeros((L,), jnp.float32)
        ones = jnp.full((L,), 1.0, jnp.float32)

        cp_p = pltpu.async_copy(
            p_hbm.at[pl.ds(tid * (_EPT // _CHUNK), _EPT // _CHUNK)],
            p_v, sem_p)
        cp_i = pltpu.async_copy(
            idx_hbm.at[pl.ds(tid * (_EPT // _CHUNK), _EPT // _CHUNK)],
            idx_v, sem_i)

        def fill_body(i, carry):
            j = i // (_CHUNK // L)
            k = i % (_CHUNK // L)
            ones_v[j, pl.ds(k * L, L)] = ones
            return carry

        lax.fori_loop(0, _EPT // L, fill_body, 0)

        def zseg_body(i, carry):
            seg_v[pl.ds(i * L, L)] = zeros
            return carry

        lax.fori_loop(0, _RPT // L, zseg_body, 0)
        pltpu.sync_copy(seg_v, s_sh.at[pl.ds(tid * _RPT, _RPT)])
        pltpu.sync_copy(seg_v, c_sh.at[pl.ds(tid * _RPT, _RPT)])
        cp_p.wait()
        cp_i.wait()
        plsc.subcore_barrier()

        descs = []
        for j in range(_EPT // _CHUNK):
            descs.append(pltpu.async_copy(
                p_v.at[j], s_sh.at[idx_v.at[j]], sem_sc, add=True))
            descs.append(pltpu.async_copy(
                ones_v.at[j], c_sh.at[idx_v.at[j]], sem_sc, add=True))
        for d in descs:
            d.wait()
        plsc.subcore_barrier()

        pltpu.sync_copy(s_sh.at[pl.ds(tid * _RPT, _RPT)], seg_v)
        pltpu.sync_copy(c_sh.at[pl.ds(tid * _RPT, _RPT)], cnt_v)

        def score_body(i, mx):
            cv = jnp.maximum(cnt_v[pl.ds(i * L, L)], 1.0)
            sc = seg_v[pl.ds(i * L, L)] / cv * INV_SQRT_D
            seg_v[pl.ds(i * L, L)] = sc
            cnt_v[pl.ds(i * L, L)] = cv
            return jnp.maximum(mx, sc)

        mx = lax.fori_loop(0, _RPT // L, score_body,
                           jnp.full((L,), -1e30, jnp.float32))
        lane0 = lax.iota(jnp.int32, L) == 0
        neg_v = jnp.full((L,), -1e30, jnp.float32)
        part_v[...] = jnp.where(lane0, jnp.full((L,), jnp.max(mx)), neg_v)
        pltpu.sync_copy(part_v, mx_sh.at[pl.ds(tid * L, L)])
        plsc.subcore_barrier()
        pltpu.sync_copy(mx_sh, red_v)

        def gmax_body(i, mx2):
            return jnp.maximum(mx2, red_v[pl.ds(i * L, L)])

        gmx = lax.fori_loop(0, _NT, gmax_body, neg_v)
        mx_s = jnp.full((L,), jnp.max(gmx))

        def exp_body(i, sm):
            e = jnp.exp(seg_v[pl.ds(i * L, L)] - mx_s)
            seg_v[pl.ds(i * L, L)] = e
            return sm + e

        sm = lax.fori_loop(0, _RPT // L, exp_body, zeros)
        part_v[...] = jnp.where(lane0, jnp.full((L,), jnp.sum(sm)), zeros)
        pltpu.sync_copy(part_v, sm_sh.at[pl.ds(tid * L, L)])
        plsc.subcore_barrier()
        pltpu.sync_copy(sm_sh, red_v)

        def gsum_body(i, acc):
            return acc + red_v[pl.ds(i * L, L)]

        gsm = lax.fori_loop(0, _NT, gsum_body, zeros)
        inv_sum = ones / jnp.full((L,), jnp.sum(gsm))

        def w_body(i, carry):
            seg_v[pl.ds(i * L, L)] = (seg_v[pl.ds(i * L, L)] * inv_sum
                                      / cnt_v[pl.ds(i * L, L)])
            return carry

        lax.fori_loop(0, _RPT // L, w_body, 0)
        pltpu.sync_copy(seg_v, w_sh.at[pl.ds(tid * _RPT, _RPT)])
        plsc.subcore_barrier()

        pltpu.sync_copy(w_sh, w_v)

        def gather_body(i, carry):
            j = i // (_CHUNK // L)
            k = i % (_CHUNK // L)
            iv = idx_v[j, pl.ds(k * L, L)]
            o_v[pl.ds(i * L, L)] = plsc.load_gather(w_v, [iv])
            return carry

        lax.fori_loop(0, _EPT // L, gather_body, 0)
        pltpu.sync_copy(o_v, coeff_hbm.at[pl.ds(tid * _EPT, _EPT)])


_OUT_G = 8
_OUT_B = N_EMB // _OUT_G


def _out_body(c_ref, e_ref, w_ref, o_ref, u_scr):
    @pl.when(pl.program_id(0) == 0)
    def _():
        u_scr[...] = jnp.zeros_like(u_scr)

    cb = c_ref[...].reshape(1, _OUT_B)
    u_scr[...] += jnp.dot(cb, e_ref[...],
                          preferred_element_type=jnp.float32,
                          precision=lax.Precision.DEFAULT)

    @pl.when(pl.program_id(0) == _OUT_G - 1)
    def _():
        o_ref[...] = jnp.dot(u_scr[...], w_ref[...],
                             preferred_element_type=jnp.float32,
                             precision=lax.Precision.DEFAULT).reshape(D)


_out_k = pl.pallas_call(
    _out_body,
    grid=(_OUT_G,),
    in_specs=[
        pl.BlockSpec((_OUT_B,), lambda i: (i,)),
        pl.BlockSpec((_OUT_B, D), lambda i: (i, 0)),
        pl.BlockSpec((D, D), lambda i: (0, 0)),
    ],
    out_specs=pl.BlockSpec((D,), lambda i: (0,)),
    out_shape=jax.ShapeDtypeStruct((D,), jnp.float32),
    scratch_shapes=[pltpu.VMEM((1, D), jnp.float32)],
)


@jax.jit
def kernel(out_embs, to_indices, query_rel, W):
    p = _proj(query_rel, W, out_embs)
    idx = to_indices.astype(jnp.int32)
    coeff = _sc_middle(p.reshape(N_EMB // _CHUNK, _CHUNK),
                       idx.reshape(N_EMB // _CHUNK, _CHUNK))
    return _out_k(coeff, out_embs, W)


# speedup vs baseline: 1.0226x; 1.0226x over previous
import functools

import jax
import jax.numpy as jnp
from jax import lax
from jax.experimental import pallas as pl
from jax.experimental.pallas import tpu as pltpu
from jax.experimental.pallas import tpu_sc as plsc

N_EMB = 8192
N_REL = 4096
D = 1024
L = 16
INV_SQRT_D = 1.0 / (D ** 0.5)


_PROJ_G = 4
_PROJ_B = N_EMB // _PROJ_G


def _proj_body(q_ref, w_ref, e_ref, p_ref, v_scr):
    @pl.when(pl.program_id(0) == 0)
    def _():
        q2d = q_ref[...].reshape(1, D)
        v_scr[...] = lax.dot_general(
            q2d, w_ref[...], (((1,), (1,)), ((), ())),
            preferred_element_type=jnp.float32,
            precision=lax.Precision.DEFAULT)

    pb = lax.dot_general(
        v_scr[...], e_ref[...], (((1,), (1,)), ((), ())),
        preferred_element_type=jnp.float32,
        precision=lax.Precision.DEFAULT)
    p_ref[...] = pb.reshape(_PROJ_B)


_proj = pl.pallas_call(
    _proj_body,
    grid=(_PROJ_G,),
    in_specs=[
        pl.BlockSpec((D,), lambda i: (0,)),
        pl.BlockSpec((D, D), lambda i: (0, 0)),
        pl.BlockSpec((_PROJ_B, D), lambda i: (i, 0)),
    ],
    out_specs=pl.BlockSpec((_PROJ_B,), lambda i: (i,)),
    out_shape=jax.ShapeDtypeStruct((N_EMB,), jnp.float32),
    scratch_shapes=[pltpu.VMEM((1, D), jnp.float32)],
)


_sc_mesh = plsc.VectorSubcoreMesh(
    core_axis_name="c", subcore_axis_name="s", num_cores=2, num_subcores=16)

_NT = 16
_EPT = N_EMB // _NT
_RPT = N_REL // _NT
_CHUNK = 128


@functools.partial(
    pl.kernel,
    out_type=jax.ShapeDtypeStruct((N_EMB // _CHUNK, _CHUNK), jnp.float32),
    mesh=_sc_mesh,
    scratch_types=[
        pltpu.VMEM((_EPT // _CHUNK, _CHUNK), jnp.float32),
        pltpu.VMEM((_EPT // _CHUNK, _CHUNK), jnp.int32),
        pltpu.VMEM((_EPT // _CHUNK, _CHUNK), jnp.float32),
        pltpu.VMEM((_RPT,), jnp.float32),
        pltpu.VMEM((_RPT,), jnp.float32),
        pltpu.VMEM((L,), jnp.float32),
        pltpu.VMEM((_NT * L,), jnp.float32),
        pltpu.VMEM((_EPT // _CHUNK, _CHUNK), jnp.float32),
        pltpu.VMEM_SHARED((N_REL,), jnp.float32),
        pltpu.VMEM_SHARED((N_REL,), jnp.float32),
        pltpu.VMEM_SHARED((N_REL,), jnp.float32),
        pltpu.VMEM_SHARED((_NT * L,), jnp.float32),
        pltpu.VMEM_SHARED((_NT * L,), jnp.float32),
        pltpu.SemaphoreType.DMA,
        pltpu.SemaphoreType.DMA,
        pltpu.SemaphoreType.DMA,
    ],
    compiler_params=pltpu.CompilerParams(needs_layout_passes=False),
)
def _sc_middle(p_hbm, idx_hbm, coeff_hbm,
               p_v, idx_v, ones_v, seg_v, cnt_v, part_v, red_v, o2_v,
               s_sh, c_sh, w_sh, mx_sh, sm_sh, sem_p, sem_i, sem_sc):
    cid = lax.axis_index("c")
    sid = lax.axis_index("s")

    @pl.when(cid == 0)
    def _():
        tid = sid
        zeros = jnp.zeros((L,), jnp.float32)
        ones = jnp.full((L,), 1.0, jnp.float32)

        cp_p = pltpu.async_copy(
            p_hbm.at[pl.ds(tid * (_EPT // _CHUNK), _EPT // _CHUNK)],
            p_v, sem_p)
        cp_i = pltpu.async_copy(
            idx_hbm.at[pl.ds(tid * (_EPT // _CHUNK), _EPT // _CHUNK)],
            idx_v, sem_i)

        def fill_body(i, carry):
            j = i // (_CHUNK // L)
            k = i % (_CHUNK // L)
            ones_v[j, pl.ds(k * L, L)] = ones
            return carry

        lax.fori_loop(0, _EPT // L, fill_body, 0)

        def zseg_body(i, carry):
            seg_v[pl.ds(i * L, L)] = zeros
            return carry

        lax.fori_loop(0, _RPT // L, zseg_body, 0)
        pltpu.sync_copy(seg_v, s_sh.at[pl.ds(tid * _RPT, _RPT)])
        pltpu.sync_copy(seg_v, c_sh.at[pl.ds(tid * _RPT, _RPT)])
        cp_p.wait()
        cp_i.wait()
        plsc.subcore_barrier()

        descs = []
        for j in range(_EPT // _CHUNK):
            descs.append(pltpu.async_copy(
                p_v.at[j], s_sh.at[idx_v.at[j]], sem_sc, add=True))
            descs.append(pltpu.async_copy(
                ones_v.at[j], c_sh.at[idx_v.at[j]], sem_sc, add=True))
        for d in descs:
            d.wait()
        plsc.subcore_barrier()

        pltpu.sync_copy(s_sh.at[pl.ds(tid * _RPT, _RPT)], seg_v)
        pltpu.sync_copy(c_sh.at[pl.ds(tid * _RPT, _RPT)], cnt_v)

        def score_body(i, mx):
            cv = jnp.maximum(cnt_v[pl.ds(i * L, L)], 1.0)
            sc = seg_v[pl.ds(i * L, L)] / cv * INV_SQRT_D
            seg_v[pl.ds(i * L, L)] = sc
            cnt_v[pl.ds(i * L, L)] = cv
            return jnp.maximum(mx, sc)

        mx = lax.fori_loop(0, _RPT // L, score_body,
                           jnp.full((L,), -1e30, jnp.float32))
        lane0 = lax.iota(jnp.int32, L) == 0
        neg_v = jnp.full((L,), -1e30, jnp.float32)
        part_v[...] = jnp.where(lane0, jnp.full((L,), jnp.max(mx)), neg_v)
        pltpu.sync_copy(part_v, mx_sh.at[pl.ds(tid * L, L)])
        plsc.subcore_barrier()
        pltpu.sync_copy(mx_sh, red_v)

        def gmax_body(i, mx2):
            return jnp.maximum(mx2, red_v[pl.ds(i * L, L)])

        gmx = lax.fori_loop(0, _NT, gmax_body, neg_v)
        mx_s = jnp.full((L,), jnp.max(gmx))

        def exp_body(i, sm):
            e = jnp.exp(seg_v[pl.ds(i * L, L)] - mx_s)
            seg_v[pl.ds(i * L, L)] = e
            return sm + e

        sm = lax.fori_loop(0, _RPT // L, exp_body, zeros)
        part_v[...] = jnp.where(lane0, jnp.full((L,), jnp.sum(sm)), zeros)
        pltpu.sync_copy(part_v, sm_sh.at[pl.ds(tid * L, L)])
        plsc.subcore_barrier()
        pltpu.sync_copy(sm_sh, red_v)

        def gsum_body(i, acc):
            return acc + red_v[pl.ds(i * L, L)]

        gsm = lax.fori_loop(0, _NT, gsum_body, zeros)
        inv_sum = ones / jnp.full((L,), jnp.sum(gsm))

        def w_body(i, carry):
            seg_v[pl.ds(i * L, L)] = (seg_v[pl.ds(i * L, L)] * inv_sum
                                      / cnt_v[pl.ds(i * L, L)])
            return carry

        lax.fori_loop(0, _RPT // L, w_body, 0)
        pltpu.sync_copy(seg_v, w_sh.at[pl.ds(tid * _RPT, _RPT)])
        plsc.subcore_barrier()

        gdescs = []
        for j in range(_EPT // _CHUNK):
            gdescs.append(pltpu.async_copy(
                w_sh.at[idx_v.at[j]], o2_v.at[j], sem_sc))
        for d in gdescs:
            d.wait()
        pltpu.sync_copy(o2_v, coeff_hbm.at[pl.ds(tid * (_EPT // _CHUNK),
                                                 _EPT // _CHUNK)])


_OUT_G = 4
_OUT_B = N_EMB // _OUT_G


def _out_body(c_ref, e_ref, w_ref, o_ref, u_scr):
    @pl.when(pl.program_id(0) == 0)
    def _():
        u_scr[...] = jnp.zeros_like(u_scr)

    cb = c_ref[...].reshape(1, _OUT_B)
    u_scr[...] += jnp.dot(cb, e_ref[...],
                          preferred_element_type=jnp.float32,
                          precision=lax.Precision.DEFAULT)

    @pl.when(pl.program_id(0) == _OUT_G - 1)
    def _():
        o_ref[...] = jnp.dot(u_scr[...], w_ref[...],
                             preferred_element_type=jnp.float32,
                             precision=lax.Precision.DEFAULT).reshape(D)


_out_k = pl.pallas_call(
    _out_body,
    grid=(_OUT_G,),
    in_specs=[
        pl.BlockSpec((_OUT_B,), lambda i: (i,)),
        pl.BlockSpec((_OUT_B, D), lambda i: (i, 0)),
        pl.BlockSpec((D, D), lambda i: (0, 0)),
    ],
    out_specs=pl.BlockSpec((D,), lambda i: (0,)),
    out_shape=jax.ShapeDtypeStruct((D,), jnp.float32),
    scratch_shapes=[pltpu.VMEM((1, D), jnp.float32)],
)


@jax.jit
def kernel(out_embs, to_indices, query_rel, W):
    p = _proj(query_rel, W, out_embs)
    idx = to_indices.astype(jnp.int32)
    coeff = _sc_middle(p.reshape(N_EMB // _CHUNK, _CHUNK),
                       idx.reshape(N_EMB // _CHUNK, _CHUNK))
    return _out_k(coeff.reshape(N_EMB), out_embs, W)
